# baseline (speedup 1.0000x reference)
import functools

import jax
import jax.numpy as jnp
from jax import lax
from jax.experimental import pallas as pl
from jax.experimental.pallas import tpu as pltpu

NUM_EMB = 1000
EMB_DIM = 1024
BATCH = 16384

ROWS_PER_BLK = 512
NBLK = BATCH // ROWS_PER_BLK


def _tc_body(idx_ref, table_ref, out_ref):
    i = pl.program_id(0)

    def f(j, _):
        r = idx_ref[i * ROWS_PER_BLK + j]
        out_ref[j] = table_ref[r]
        return 0

    lax.fori_loop(0, ROWS_PER_BLK, f, 0, unroll=8)


_tc_gather = pl.pallas_call(
    _tc_body,
    grid_spec=pltpu.PrefetchScalarGridSpec(
        num_scalar_prefetch=1,
        grid=(NBLK,),
        in_specs=[
            pl.BlockSpec((NUM_EMB, 8, 128), lambda i, idx: (0, 0, 0)),
        ],
        out_specs=pl.BlockSpec((ROWS_PER_BLK, 8, 128),
                               lambda i, idx: (i, 0, 0)),
    ),
    out_shape=jax.ShapeDtypeStruct((BATCH, 8, 128), jnp.float32),
)


@jax.jit
def kernel(x, pos_encoding):
    table = pos_encoding.reshape(NUM_EMB, 8, 128)
    out = _tc_gather(x, table)
    return out.reshape(BATCH, EMB_DIM)
